# baseline (speedup 1.0000x reference)
import functools

import jax
import jax.numpy as jnp
from jax import lax
from jax.experimental import pallas as pl
from jax.experimental.pallas import tpu as pltpu
from jax.experimental.pallas import tpu_sc as plsc

_VOCAB = 100000
_DIM = 64
_B = 16384
_K = 5

_NC = 2
_NS = 16
_NW = _NC * _NS
_BPW = _B // _NW
_S = 64
_NCHUNK = _BPW // _S
_G = _S // 16


@functools.cache
def _make_sc_logits():
    mesh = plsc.VectorSubcoreMesh(core_axis_name="c", subcore_axis_name="s")

    @functools.partial(
        pl.kernel,
        mesh=mesh,
        compiler_params=pltpu.CompilerParams(
            needs_layout_passes=False, use_tc_tiling_on_sc=False),
        out_type=jax.ShapeDtypeStruct((6 * _B,), jnp.float32),
        scratch_types=[
            pltpu.VMEM((_BPW,), jnp.int32),
            pltpu.VMEM((_BPW,), jnp.int32),
            pltpu.VMEM((_K * _BPW,), jnp.int32),
            pltpu.VMEM((_S, _DIM), jnp.bfloat16),
            pltpu.VMEM((_S, _DIM), jnp.bfloat16),
            pltpu.VMEM((_K * _S, _DIM), jnp.bfloat16),
            pltpu.VMEM((_S, _DIM), jnp.bfloat16),
            pltpu.VMEM((_S, _DIM), jnp.bfloat16),
            pltpu.VMEM((_K * _S, _DIM), jnp.bfloat16),
            pltpu.VMEM((6, _BPW), jnp.float32),
            pltpu.SemaphoreType.DMA,
            pltpu.SemaphoreType.DMA,
            pltpu.SemaphoreType.DMA,
        ],
    )
    def sc_logits(cen_hbm, ctx_hbm, neg_hbm, win_hbm, wout_hbm, out_hbm,
                  idxc, idxx, idxn,
                  crA, xrA, nrA, crB, xrB, nrB,
                  lbuf, semi, semA, semB):
        wid = lax.axis_index("s") * _NC + lax.axis_index("c")
        base = wid * _BPW
        iota = lax.iota(jnp.int32, 16)

        cpi = [pltpu.async_copy(cen_hbm.at[pl.ds(base, _BPW)], idxc, semi),
               pltpu.async_copy(ctx_hbm.at[pl.ds(base, _BPW)], idxx, semi),
               pltpu.async_copy(neg_hbm.at[pl.ds(base * _K, _K * _BPW)],
                                idxn, semi)]
        for cp in cpi:
            cp.wait()

        def gather_bufs(t, cr, xr, nr, sem):
            toff = pl.multiple_of(t * _S, _S)
            return [
                pltpu.async_copy(win_hbm.at[idxc.at[pl.ds(toff, _S)]],
                                 cr, sem),
                pltpu.async_copy(wout_hbm.at[idxx.at[pl.ds(toff, _S)]],
                                 xr, sem),
                pltpu.async_copy(wout_hbm.at[idxn.at[pl.ds(toff * _K,
                                                           _K * _S)]],
                                 nr, sem),
            ]

        def wait_bufs(t, cr, xr, nr, sem):
            toff = pl.multiple_of(t * _S, _S)
            pltpu.make_async_copy(win_hbm.at[idxc.at[pl.ds(toff, _S)]],
                                  cr, sem).wait()
            pltpu.make_async_copy(wout_hbm.at[idxx.at[pl.ds(toff, _S)]],
                                  xr, sem).wait()
            pltpu.make_async_copy(wout_hbm.at[idxn.at[pl.ds(toff * _K,
                                                            _K * _S)]],
                                  nr, sem).wait()

        def compute_chunk(t, cr, xr, nr):
            toff = pl.multiple_of(t * _S, _S)

            def g_body(g, carry):
                s0 = pl.multiple_of(g * 16, 16)
                accs = [jnp.zeros((16,), jnp.float32) for _ in range(6)]

                def row_f32(ref, r):
                    out = []
                    for k in range(_DIM // 32):
                        out.extend(plsc.unpack(
                            ref[r, pl.ds(k * 32, 32)],
                            format=plsc.PackFormat.INTERLEAVED,
                            preferred_element_type=jnp.float32))
                    return out

                for l in range(16):
                    s = s0 + l
                    lane = iota == l
                    cvs = row_f32(cr, s)
                    for j in range(6):
                        if j == 0:
                            rvs = row_f32(xr, s)
                        else:
                            rvs = row_f32(nr, s * _K + (j - 1))
                        p = cvs[0] * rvs[0]
                        for k in range(1, _DIM // 16):
                            p = p + cvs[k] * rvs[k]
                        r = jnp.sum(p)
                        accs[j] = jnp.where(lane, r, accs[j])
                for j in range(6):
                    lbuf[j, pl.ds(toff + s0, 16)] = accs[j]
                return carry

            lax.fori_loop(0, _G, g_body, 0)

        gather_bufs(0, crA, xrA, nrA, semA)

        def pair_body(pr, carry):
            t0 = pr * 2
            t1 = t0 + 1
            gather_bufs(t1, crB, xrB, nrB, semB)
            wait_bufs(t0, crA, xrA, nrA, semA)
            compute_chunk(t0, crA, xrA, nrA)

            @pl.when(pr < _NCHUNK // 2 - 1)
            def _():
                gather_bufs(t0 + 2, crA, xrA, nrA, semA)

            wait_bufs(t1, crB, xrB, nrB, semB)
            compute_chunk(t1, crB, xrB, nrB)
            return carry

        lax.fori_loop(0, _NCHUNK // 2, pair_body, 0)

        for j in range(6):
            obase = pl.multiple_of(j * _B + base, _BPW)
            pltpu.sync_copy(lbuf.at[j], out_hbm.at[pl.ds(obase, _BPW)])

    return sc_logits


def _bce_body(x_ref, o_ref):
    x = x_ref[...]
    pos_rows = _B // 128
    lbl = (lax.broadcasted_iota(jnp.int32, x.shape, 0) < pos_rows
           ).astype(jnp.float32)
    v = jnp.maximum(x, 0.0) - x * lbl + jnp.log(1.0 + jnp.exp(-jnp.abs(x)))
    o_ref[0, 0] = jnp.sum(v) / (6.0 * _B)


def kernel(center, context, negatives, W_in, W_out):
    cen = center.astype(jnp.int32)
    ctx = context.reshape(_B).astype(jnp.int32)
    neg = negatives.reshape(_B * _K).astype(jnp.int32)
    wi_bf = (W_in + 0.0 * jnp.round(W_in)).astype(jnp.bfloat16)
    wo_bf = (W_out + 0.0 * jnp.round(W_out)).astype(jnp.bfloat16)
    logits = _make_sc_logits()(cen, ctx, neg, wi_bf, wo_bf)
    loss = pl.pallas_call(
        _bce_body,
        out_shape=jax.ShapeDtypeStruct((1, 1), jnp.float32),
        out_specs=pl.BlockSpec(memory_space=pltpu.SMEM),
    )(logits.reshape(6 * _B // 128, 128))
    return loss[0, 0]


# speedup vs baseline: 1.2184x; 1.2184x over previous
import functools

import jax
import jax.numpy as jnp
from jax import lax
from jax.experimental import pallas as pl
from jax.experimental.pallas import tpu as pltpu
from jax.experimental.pallas import tpu_sc as plsc

_VOCAB = 100000
_DIM = 64
_B = 16384
_K = 5

_NC = 2
_NS = 16
_NW = _NC * _NS
_BPW = _B // _NW
_S = 64
_NCHUNK = _BPW // _S
_G = _S // 16


@functools.cache
def _make_sc_logits():
    mesh = plsc.VectorSubcoreMesh(core_axis_name="c", subcore_axis_name="s")

    @functools.partial(
        pl.kernel,
        mesh=mesh,
        compiler_params=pltpu.CompilerParams(
            needs_layout_passes=False, use_tc_tiling_on_sc=False),
        out_type=jax.ShapeDtypeStruct((6 * _B,), jnp.float32),
        scratch_types=[
            pltpu.VMEM((_BPW,), jnp.int32),
            pltpu.VMEM((_BPW,), jnp.int32),
            pltpu.VMEM((_K * _BPW,), jnp.int32),
            pltpu.VMEM((_S, _DIM), jnp.bfloat16),
            pltpu.VMEM((_S, _DIM), jnp.bfloat16),
            pltpu.VMEM((_K * _S, _DIM), jnp.bfloat16),
            pltpu.VMEM((_S, _DIM), jnp.bfloat16),
            pltpu.VMEM((_S, _DIM), jnp.bfloat16),
            pltpu.VMEM((_K * _S, _DIM), jnp.bfloat16),
            pltpu.VMEM((6, _BPW), jnp.float32),
            pltpu.SemaphoreType.DMA,
            pltpu.SemaphoreType.DMA,
            pltpu.SemaphoreType.DMA,
        ],
    )
    def sc_logits(cen_hbm, ctx_hbm, neg_hbm, win_hbm, wout_hbm, out_hbm,
                  idxc, idxx, idxn,
                  crA, xrA, nrA, crB, xrB, nrB,
                  lbuf, semi, semA, semB):
        wid = lax.axis_index("s") * _NC + lax.axis_index("c")
        base = wid * _BPW
        iota = lax.iota(jnp.int32, 16)

        cpi = [pltpu.async_copy(cen_hbm.at[pl.ds(base, _BPW)], idxc, semi),
               pltpu.async_copy(ctx_hbm.at[pl.ds(base, _BPW)], idxx, semi),
               pltpu.async_copy(neg_hbm.at[pl.ds(base * _K, _K * _BPW)],
                                idxn, semi)]
        for cp in cpi:
            cp.wait()

        def gather_bufs(t, cr, xr, nr, sem):
            toff = pl.multiple_of(t * _S, _S)
            return [
                pltpu.async_copy(win_hbm.at[idxc.at[pl.ds(toff, _S)]],
                                 cr, sem),
                pltpu.async_copy(wout_hbm.at[idxx.at[pl.ds(toff, _S)]],
                                 xr, sem),
                pltpu.async_copy(wout_hbm.at[idxn.at[pl.ds(toff * _K,
                                                           _K * _S)]],
                                 nr, sem),
            ]

        def wait_bufs(t, cr, xr, nr, sem):
            toff = pl.multiple_of(t * _S, _S)
            pltpu.make_async_copy(win_hbm.at[idxc.at[pl.ds(toff, _S)]],
                                  cr, sem).wait()
            pltpu.make_async_copy(wout_hbm.at[idxx.at[pl.ds(toff, _S)]],
                                  xr, sem).wait()
            pltpu.make_async_copy(wout_hbm.at[idxn.at[pl.ds(toff * _K,
                                                            _K * _S)]],
                                  nr, sem).wait()

        def compute_chunk(t, cr, xr, nr):
            toff = pl.multiple_of(t * _S, _S)

            def g_body(g, carry):
                s0 = pl.multiple_of(g * 16, 16)
                accs = [jnp.zeros((16,), jnp.float32) for _ in range(6)]

                def row_f32(ref, r):
                    out = []
                    for k in range(_DIM // 32):
                        out.extend(plsc.unpack(
                            ref[r, pl.ds(k * 32, 32)],
                            format=plsc.PackFormat.INTERLEAVED,
                            preferred_element_type=jnp.float32))
                    return out

                for l in range(16):
                    s = s0 + l
                    lane = iota == l
                    cvs = row_f32(cr, s)
                    for j in range(6):
                        if j == 0:
                            rvs = row_f32(xr, s)
                        else:
                            rvs = row_f32(nr, s * _K + (j - 1))
                        p = cvs[0] * rvs[0]
                        for k in range(1, _DIM // 16):
                            p = p + cvs[k] * rvs[k]
                        r = jnp.sum(p)
                        accs[j] = jnp.where(lane, r, accs[j])
                for j in range(6):
                    lbuf[j, pl.ds(toff + s0, 16)] = accs[j]
                return carry

            lax.fori_loop(0, _G, g_body, 0)

        gather_bufs(0, crA, xrA, nrA, semA)

        def pair_body(pr, carry):
            t0 = pr * 2
            t1 = t0 + 1
            gather_bufs(t1, crB, xrB, nrB, semB)
            wait_bufs(t0, crA, xrA, nrA, semA)
            compute_chunk(t0, crA, xrA, nrA)

            @pl.when(pr < _NCHUNK // 2 - 1)
            def _():
                gather_bufs(t0 + 2, crA, xrA, nrA, semA)

            wait_bufs(t1, crB, xrB, nrB, semB)
            compute_chunk(t1, crB, xrB, nrB)
            return carry

        lax.fori_loop(0, _NCHUNK // 2, pair_body, 0)

        for j in range(6):
            obase = pl.multiple_of(j * _B + base, _BPW)
            pltpu.sync_copy(lbuf.at[j], out_hbm.at[pl.ds(obase, _BPW)])

    return sc_logits


def _cast_body(x_ref, o_ref):
    o_ref[...] = x_ref[...].astype(jnp.bfloat16)


def _tc_cast_bf16(w):
    rows = w.shape[0]
    blk = 4000
    return pl.pallas_call(
        _cast_body,
        grid=(rows // blk,),
        in_specs=[pl.BlockSpec((blk, _DIM), lambda i: (i, 0))],
        out_specs=pl.BlockSpec((blk, _DIM), lambda i: (i, 0)),
        out_shape=jax.ShapeDtypeStruct((rows, _DIM), jnp.bfloat16),
    )(w)


def _bce_body(x_ref, o_ref):
    x = x_ref[...]
    pos_rows = _B // 128
    lbl = (lax.broadcasted_iota(jnp.int32, x.shape, 0) < pos_rows
           ).astype(jnp.float32)
    v = jnp.maximum(x, 0.0) - x * lbl + jnp.log(1.0 + jnp.exp(-jnp.abs(x)))
    o_ref[0, 0] = jnp.sum(v) / (6.0 * _B)


def kernel(center, context, negatives, W_in, W_out):
    cen = center.astype(jnp.int32)
    ctx = context.reshape(_B).astype(jnp.int32)
    neg = negatives.reshape(_B * _K).astype(jnp.int32)
    wi_bf = _tc_cast_bf16(W_in)
    wo_bf = _tc_cast_bf16(W_out)
    logits = _make_sc_logits()(cen, ctx, neg, wi_bf, wo_bf)
    loss = pl.pallas_call(
        _bce_body,
        out_shape=jax.ShapeDtypeStruct((1, 1), jnp.float32),
        out_specs=pl.BlockSpec(memory_space=pltpu.SMEM),
    )(logits.reshape(6 * _B // 128, 128))
    return loss[0, 0]


# speedup vs baseline: 2.2522x; 1.8485x over previous
import functools

import jax
import jax.numpy as jnp
from jax import lax
from jax.experimental import pallas as pl
from jax.experimental.pallas import tpu as pltpu
from jax.experimental.pallas import tpu_sc as plsc

_VOCAB = 100000
_DIM = 64
_B = 16384
_K = 5

_NC = 2
_NS = 16
_NW = _NC * _NS
_BPW = _B // _NW
_S = 64
_NCHUNK = _BPW // _S
_G = _S // 16


@functools.cache
def _make_sc_logits():
    mesh = plsc.VectorSubcoreMesh(core_axis_name="c", subcore_axis_name="s")

    @functools.partial(
        pl.kernel,
        mesh=mesh,
        compiler_params=pltpu.CompilerParams(
            needs_layout_passes=False, use_tc_tiling_on_sc=False),
        out_type=jax.ShapeDtypeStruct((6 * _B,), jnp.float32),
        scratch_types=[
            pltpu.VMEM((_BPW,), jnp.int32),
            pltpu.VMEM((_BPW,), jnp.int32),
            pltpu.VMEM((_K * _BPW,), jnp.int32),
            pltpu.VMEM((_S, _DIM), jnp.float32),
            pltpu.VMEM((_S, _DIM), jnp.float32),
            pltpu.VMEM((_K * _S, _DIM), jnp.float32),
            pltpu.VMEM((_S, _DIM), jnp.float32),
            pltpu.VMEM((_S, _DIM), jnp.float32),
            pltpu.VMEM((_K * _S, _DIM), jnp.float32),
            pltpu.VMEM((6, _BPW), jnp.float32),
            pltpu.SemaphoreType.DMA,
            pltpu.SemaphoreType.DMA,
            pltpu.SemaphoreType.DMA,
        ],
    )
    def sc_logits(cen_hbm, ctx_hbm, neg_hbm, win_hbm, wout_hbm, out_hbm,
                  idxc, idxx, idxn,
                  crA, xrA, nrA, crB, xrB, nrB,
                  lbuf, semi, semA, semB):
        wid = lax.axis_index("s") * _NC + lax.axis_index("c")
        base = wid * _BPW
        iota = lax.iota(jnp.int32, 16)

        cpi = [pltpu.async_copy(cen_hbm.at[pl.ds(base, _BPW)], idxc, semi),
               pltpu.async_copy(ctx_hbm.at[pl.ds(base, _BPW)], idxx, semi),
               pltpu.async_copy(neg_hbm.at[pl.ds(base * _K, _K * _BPW)],
                                idxn, semi)]
        for cp in cpi:
            cp.wait()

        def gather_bufs(t, cr, xr, nr, sem):
            toff = pl.multiple_of(t * _S, _S)
            return [
                pltpu.async_copy(win_hbm.at[idxc.at[pl.ds(toff, _S)]],
                                 cr, sem),
                pltpu.async_copy(wout_hbm.at[idxx.at[pl.ds(toff, _S)]],
                                 xr, sem),
                pltpu.async_copy(wout_hbm.at[idxn.at[pl.ds(toff * _K,
                                                           _K * _S)]],
                                 nr, sem),
            ]

        def wait_bufs(t, cr, xr, nr, sem):
            toff = pl.multiple_of(t * _S, _S)
            pltpu.make_async_copy(win_hbm.at[idxc.at[pl.ds(toff, _S)]],
                                  cr, sem).wait()
            pltpu.make_async_copy(wout_hbm.at[idxx.at[pl.ds(toff, _S)]],
                                  xr, sem).wait()
            pltpu.make_async_copy(wout_hbm.at[idxn.at[pl.ds(toff * _K,
                                                            _K * _S)]],
                                  nr, sem).wait()

        def compute_chunk(t, cr, xr, nr):
            toff = pl.multiple_of(t * _S, _S)

            def g_body(g, carry):
                s0 = pl.multiple_of(g * 16, 16)
                accs = [jnp.zeros((16,), jnp.float32) for _ in range(6)]
                for l in range(16):
                    s = s0 + l
                    lane = iota == l
                    cvs = [cr[s, pl.ds(k * 16, 16)]
                           for k in range(_DIM // 16)]
                    for j in range(6):
                        if j == 0:
                            rvs = [xr[s, pl.ds(k * 16, 16)]
                                   for k in range(_DIM // 16)]
                        else:
                            rvs = [nr[s * _K + (j - 1), pl.ds(k * 16, 16)]
                                   for k in range(_DIM // 16)]
                        p = cvs[0] * rvs[0]
                        for k in range(1, _DIM // 16):
                            p = p + cvs[k] * rvs[k]
                        r = jnp.sum(p)
                        accs[j] = jnp.where(lane, r, accs[j])
                for j in range(6):
                    lbuf[j, pl.ds(toff + s0, 16)] = accs[j]
                return carry

            lax.fori_loop(0, _G, g_body, 0)

        gather_bufs(0, crA, xrA, nrA, semA)

        def pair_body(pr, carry):
            t0 = pr * 2
            t1 = t0 + 1
            gather_bufs(t1, crB, xrB, nrB, semB)
            wait_bufs(t0, crA, xrA, nrA, semA)
            compute_chunk(t0, crA, xrA, nrA)

            @pl.when(pr < _NCHUNK // 2 - 1)
            def _():
                gather_bufs(t0 + 2, crA, xrA, nrA, semA)

            wait_bufs(t1, crB, xrB, nrB, semB)
            compute_chunk(t1, crB, xrB, nrB)
            return carry

        lax.fori_loop(0, _NCHUNK // 2, pair_body, 0)

        cpo = []
        for j in range(6):
            obase = pl.multiple_of(j * _B + base, _BPW)
            cpo.append(pltpu.async_copy(
                lbuf.at[j], out_hbm.at[pl.ds(obase, _BPW)], semi))
        for cp in cpo:
            cp.wait()

    return sc_logits


def _bce_body(x_ref, o_ref):
    x = x_ref[...]
    pos_rows = _B // 128
    lbl = (lax.broadcasted_iota(jnp.int32, x.shape, 0) < pos_rows
           ).astype(jnp.float32)
    v = jnp.maximum(x, 0.0) - x * lbl + jnp.log(1.0 + jnp.exp(-jnp.abs(x)))
    o_ref[0, 0] = jnp.sum(v) / (6.0 * _B)


def kernel(center, context, negatives, W_in, W_out):
    cen = center.astype(jnp.int32)
    ctx = context.reshape(_B).astype(jnp.int32)
    neg = negatives.reshape(_B * _K).astype(jnp.int32)
    logits = _make_sc_logits()(cen, ctx, neg, W_in, W_out)
    loss = pl.pallas_call(
        _bce_body,
        out_shape=jax.ShapeDtypeStruct((1, 1), jnp.float32),
        out_specs=pl.BlockSpec(memory_space=pltpu.SMEM),
    )(logits.reshape(6 * _B // 128, 128))
    return loss[0, 0]
